# baseline (speedup 1.0000x reference)
import jax
import jax.numpy as jnp
from jax import lax
from jax.experimental import pallas as pl
from jax.experimental.pallas import tpu as pltpu
from jax.experimental.pallas import tpu_sc as plsc

VOCAB = 1000000
EMBED = 32
BATCH = 4096
SEQ = 200

_INFO = plsc.get_sparse_core_info()
_NC = _INFO.num_cores
_NS = _INFO.num_subcores
_L = _INFO.num_lanes
_NW = _NC * _NS
_BPW = BATCH // _NW
_IPW = _BPW * SEQ
_CH = 128
_NCH = _IPW // _CH

_TC_ROWS = 32768


_TCB = 8192
_TC_NB = VOCAB // _TCB
_TC_TAIL = _TC_NB * _TCB
_TC_DEPTH = 4


def _tc_body(w_ref, table_any, t_any, *scratch):
    ibufs = scratch[:_TC_DEPTH]
    obufs = scratch[_TC_DEPTH:_TC_DEPTH + 2]
    isem, osem = scratch[_TC_DEPTH + 2], scratch[_TC_DEPTH + 3]
    i = pl.program_id(0)

    def in_copy(b, k):
        return pltpu.make_async_copy(
            table_any.at[pl.ds(pl.multiple_of(b * _TCB, 8), _TCB), :],
            ibufs[k], isem.at[k])

    def out_copy(b, j):
        return pltpu.make_async_copy(
            obufs[j], t_any.at[pl.ds(pl.multiple_of(b * _TCB, 8), _TCB)],
            osem.at[j])

    @pl.when(i == 0)
    def _():
        for b in range(_TC_DEPTH):
            in_copy(jnp.int32(b), b).start()

    slot = lax.rem(i, _TC_DEPTH)
    for k in range(_TC_DEPTH):
        @pl.when(slot == k)
        def _(k=k):
            in_copy(i, k).wait()

            @pl.when(i >= 2)
            def _():
                out_copy(i - 2, k % 2).wait()

            obufs[k % 2][...] = jax.lax.dot_general(
                w_ref[...], ibufs[k][...], (((1,), (1,)), ((), ())),
                preferred_element_type=jnp.float32)[0, :]
            out_copy(i, k % 2).start()

            @pl.when(i + _TC_DEPTH < _TC_NB)
            def _():
                in_copy(i + _TC_DEPTH, k).start()

    @pl.when(i == _TC_NB - 1)
    def _():
        out_copy(i - 1, (_TC_NB - 2) % 2).wait()
        out_copy(i, (_TC_NB - 1) % 2).wait()


def _tc_tail_body(w_ref, table_ref, t_ref):
    t_ref[...] = jax.lax.dot_general(
        w_ref[...], table_ref[...], (((1,), (1,)), ((), ())),
        preferred_element_type=jnp.float32)[0, :]


def _tc_matvec(table, w_row):
    tail = jax.lax.slice(table, (_TC_TAIL, 0), (VOCAB, EMBED))
    t_tail = pl.pallas_call(
        _tc_tail_body,
        out_shape=jax.ShapeDtypeStruct((VOCAB - _TC_TAIL,), jnp.float32),
    )(w_row, tail)
    t_main = pl.pallas_call(
        _tc_body,
        grid=(_TC_NB,),
        in_specs=[
            pl.BlockSpec((1, EMBED), lambda i: (0, 0)),
            pl.BlockSpec(memory_space=pl.ANY),
        ],
        out_specs=pl.BlockSpec(memory_space=pl.ANY),
        out_shape=jax.ShapeDtypeStruct((_TC_TAIL,), jnp.float32),
        scratch_shapes=(
            [pltpu.VMEM((_TCB, EMBED), jnp.float32)
             for _ in range(_TC_DEPTH)]
            + [pltpu.VMEM((_TCB,), jnp.float32) for _ in range(2)]
            + [pltpu.SemaphoreType.DMA((_TC_DEPTH,)),
               pltpu.SemaphoreType.DMA((2,))]
        ),
    )(w_row, table)
    return jnp.concatenate([t_main, t_tail])


def _sc_body(x_hbm, params_hbm, t_hbm, out_hbm, idx_v, params_v, vals_v,
             outs_v, sem):
    wid = lax.axis_index("s") * _NC + lax.axis_index("c")
    base = wid * _IPW

    pltpu.sync_copy(x_hbm.at[pl.ds(base, _IPW)], idx_v)
    pltpu.sync_copy(params_hbm, params_v)
    bias_v = params_v[pl.ds(0, _L)]

    def fire(c, carry):
        off = pl.multiple_of(c * _CH, 8)
        pltpu.async_copy(
            t_hbm.at[idx_v.at[pl.ds(off, _CH)]], vals_v.at[pl.ds(off, _CH)],
            sem)
        return carry

    lax.fori_loop(0, _NCH, fire, 0)
    pltpu.make_async_copy(t_hbm.at[idx_v], vals_v, sem).wait()

    lanes = lax.iota(jnp.int32, _L)
    nfull = SEQ // _L
    tail_keep = lanes >= (_L - SEQ % _L)
    zero = jnp.zeros((_L,), jnp.float32)

    def elem(e, lanevec):
        off = pl.multiple_of(e * SEQ, 8)
        acc = zero
        for k in range(nfull):
            acc = acc + vals_v[pl.ds(off + k * _L, _L)]
        tail = vals_v[pl.ds(off + SEQ - _L, _L)]
        acc = acc + jnp.where(tail_keep, tail, zero)
        s = jnp.sum(acc)
        lanevec = jnp.where(lanes == e % _L, s, lanevec)

        @pl.when(e % _L == _L - 1)
        def _():
            outs_v[pl.ds(pl.multiple_of((e // _L) * _L, _L), _L)] = lanevec

        return lanevec

    lax.fori_loop(0, _BPW, elem, zero)

    one = jnp.float32(1.0)
    for k in range(_BPW // _L):
        z = outs_v[pl.ds(k * _L, _L)] + bias_v
        outs_v[pl.ds(k * _L, _L)] = one / (one + jnp.exp(-z))

    pltpu.sync_copy(outs_v, out_hbm.at[pl.ds(wid * _BPW, _BPW)])


def _sc_gather(x_flat, params, t):
    mesh = plsc.VectorSubcoreMesh(core_axis_name="c", subcore_axis_name="s")
    return pl.kernel(
        _sc_body,
        jax.ShapeDtypeStruct((BATCH,), jnp.float32),
        mesh=mesh,
        scratch_types=[
            pltpu.VMEM((_IPW,), jnp.int32),
            pltpu.VMEM((_L,), jnp.float32),
            pltpu.VMEM((_IPW,), jnp.float32),
            pltpu.VMEM((_BPW,), jnp.float32),
            pltpu.SemaphoreType.DMA,
        ],
        compiler_params=pltpu.CompilerParams(
            needs_layout_passes=False, use_tc_tiling_on_sc=False),
    )(x_flat, params, t)


@jax.jit
def _run(x, emb_table, fc_w, fc_b):
    x_flat = x.reshape(-1).astype(jnp.int32)
    w_row = fc_w.reshape(1, EMBED) * jnp.float32(1.0 / SEQ)
    params = jnp.broadcast_to(fc_b.reshape(-1), (_L,))
    t = _tc_matvec(emb_table, w_row)
    return _sc_gather(x_flat, params, t)


def kernel(x, emb_table, fc_w, fc_b):
    return _run(x, emb_table, fc_w, fc_b).reshape(BATCH, 1)


# speedup vs baseline: 1.0148x; 1.0148x over previous
import jax
import jax.numpy as jnp
from jax import lax
from jax.experimental import pallas as pl
from jax.experimental.pallas import tpu as pltpu
from jax.experimental.pallas import tpu_sc as plsc

VOCAB = 1000000
EMBED = 32
BATCH = 4096
SEQ = 200

_INFO = plsc.get_sparse_core_info()
_NC = _INFO.num_cores
_NS = _INFO.num_subcores
_L = _INFO.num_lanes
_NW = _NC * _NS
_BPW = BATCH // _NW
_IPW = _BPW * SEQ
_CH = 128
_NCH = _IPW // _CH

_TC_ROWS = 32768


def _tc_body(w_ref, table_ref, t_ref):
    t_ref[...] = jax.lax.dot_general(
        w_ref[...], table_ref[...], (((1,), (1,)), ((), ())),
        preferred_element_type=jnp.float32)[0, :]


def _tc_matvec(table, w_row):
    grid = pl.cdiv(VOCAB, _TC_ROWS)
    return pl.pallas_call(
        _tc_body,
        grid=(grid,),
        in_specs=[
            pl.BlockSpec((1, EMBED), lambda i: (0, 0)),
            pl.BlockSpec((_TC_ROWS, EMBED), lambda i: (i, 0)),
        ],
        out_specs=pl.BlockSpec((_TC_ROWS,), lambda i: (i,)),
        out_shape=jax.ShapeDtypeStruct((VOCAB,), jnp.float32),
    )(w_row, table)


def _sc_body(x_hbm, params_hbm, t_hbm, out_hbm, idx_v, params_v, vals_v,
             outs_v, sem):
    wid = lax.axis_index("s") * _NC + lax.axis_index("c")
    base = wid * _IPW

    pltpu.sync_copy(x_hbm.at[pl.ds(base, _IPW)], idx_v)
    pltpu.sync_copy(params_hbm, params_v)
    bias_v = params_v[pl.ds(0, _L)]

    def fire(c, carry):
        off = pl.multiple_of(c * _CH, 8)
        pltpu.async_copy(
            t_hbm.at[idx_v.at[pl.ds(off, _CH)]], vals_v.at[pl.ds(off, _CH)],
            sem)
        return carry

    lax.fori_loop(0, _NCH, fire, 0)
    pltpu.make_async_copy(t_hbm.at[idx_v], vals_v, sem).wait()

    lanes = lax.iota(jnp.int32, _L)
    nfull = SEQ // _L
    tail_keep = lanes >= (_L - SEQ % _L)
    zero = jnp.zeros((_L,), jnp.float32)

    def elem(e, lanevec):
        off = pl.multiple_of(e * SEQ, 8)
        acc = zero
        for k in range(nfull):
            acc = acc + vals_v[pl.ds(off + k * _L, _L)]
        tail = vals_v[pl.ds(off + SEQ - _L, _L)]
        acc = acc + jnp.where(tail_keep, tail, zero)
        s = jnp.sum(acc)
        lanevec = jnp.where(lanes == e % _L, s, lanevec)

        @pl.when(e % _L == _L - 1)
        def _():
            outs_v[pl.ds(pl.multiple_of((e // _L) * _L, _L), _L)] = lanevec

        return lanevec

    lax.fori_loop(0, _BPW, elem, zero)

    one = jnp.float32(1.0)
    for k in range(_BPW // _L):
        z = outs_v[pl.ds(k * _L, _L)] + bias_v
        outs_v[pl.ds(k * _L, _L)] = one / (one + jnp.exp(-z))

    pltpu.sync_copy(outs_v, out_hbm.at[pl.ds(wid * _BPW, _BPW)])


def _sc_gather(x_flat, params, t):
    mesh = plsc.VectorSubcoreMesh(core_axis_name="c", subcore_axis_name="s")
    return pl.kernel(
        _sc_body,
        jax.ShapeDtypeStruct((BATCH,), jnp.float32),
        mesh=mesh,
        scratch_types=[
            pltpu.VMEM((_IPW,), jnp.int32),
            pltpu.VMEM((_L,), jnp.float32),
            pltpu.VMEM((_IPW,), jnp.float32),
            pltpu.VMEM((_BPW,), jnp.float32),
            pltpu.SemaphoreType.DMA,
        ],
        compiler_params=pltpu.CompilerParams(
            needs_layout_passes=False, use_tc_tiling_on_sc=False),
    )(x_flat, params, t)


@jax.jit
def _run(x, emb_table, fc_w, fc_b):
    x_flat = x.reshape(-1).astype(jnp.int32)
    w_row = fc_w.reshape(1, EMBED) * jnp.float32(1.0 / SEQ)
    params = jnp.broadcast_to(fc_b.reshape(-1), (_L,))
    t = _tc_matvec(emb_table, w_row)
    return _sc_gather(x_flat, params, t)


def kernel(x, emb_table, fc_w, fc_b):
    return _run(x, emb_table, fc_w, fc_b).reshape(BATCH, 1)
